# baseline (speedup 1.0000x reference)
import functools

import jax
import jax.numpy as jnp
from jax import lax
from jax.experimental import pallas as pl
from jax.experimental.pallas import tpu as pltpu
from jax.experimental.pallas import tpu_sc as plsc

B = 1024
IN_DIM = 4096
OUT_DIM = 4096
FAN_IN = 8
L = 16
NC, NS = 2, 16
NW = NC * NS
RPW = B // NW
PPW = RPW // 2
G = OUT_DIM // L
PB = 4
NPASS = PPW // PB


def _tec_body(xp_hbm, idxT_hbm, wp_hbm, s_hbm, b_hbm, out_hbm,
              x0, x1, x2, x3, y0, y1, y2, y3, y4, y5, y6, y7,
              idxv, wv, sv, bv, sem):
    xr = (x0, x1, x2, x3)
    yr = (y0, y1, y2, y3, y4, y5, y6, y7)
    wid = lax.axis_index("s") * NC + lax.axis_index("c")
    pbase = wid * PPW
    pltpu.sync_copy(idxT_hbm, idxv)
    pltpu.sync_copy(wp_hbm, wv)
    pltpu.sync_copy(s_hbm, sv)
    pltpu.sync_copy(b_hbm, bv)

    def pass_body(p, carry):
        pair = pbase + p * PB
        cps = [pltpu.async_copy(xp_hbm.at[pair + j], xr[j], sem)
               for j in range(PB)]
        for c in cps:
            c.wait()

        def grp_body(g, c2):
            o = g * L
            acc = [None] * PB
            for k in range(FAN_IN):
                ivec = idxv[k, pl.ds(o, L)]
                wvec = plsc.bitcast(wv[k, pl.ds(o, L)], jnp.bfloat16)
                for j in range(PB):
                    gb = plsc.bitcast(plsc.load_gather(xr[j], [ivec]),
                                      jnp.bfloat16)
                    t = gb * wvec
                    acc[j] = t if k == 0 else acc[j] + t
            svec = sv[pl.ds(o, L)]
            bvec = bv[pl.ds(o, L)]
            for j in range(PB):
                lo, hi = plsc.unpack(acc[j],
                                     format=plsc.PackFormat.INTERLEAVED)
                for r, val in ((2 * j, lo), (2 * j + 1, hi)):
                    z = val * svec - bvec
                    yr[r][pl.ds(o, L)] = 1.0 / (1.0 + jnp.exp(-z))
            return c2

        lax.fori_loop(0, G, grp_body, 0)
        ocps = [pltpu.async_copy(yr[r], out_hbm.at[2 * pair + r], sem)
                for r in range(2 * PB)]
        for c in ocps:
            c.wait()
        return carry

    lax.fori_loop(0, NPASS, pass_body, 0)


def kernel(x, idx, w, theta, s_raw):
    xu = lax.bitcast_convert_type(x.astype(jnp.bfloat16),
                                  jnp.uint16).astype(jnp.uint32)
    xpack = lax.bitcast_convert_type(
        xu[0::2] | (xu[1::2] << 16), jnp.int32)

    idxT = jnp.asarray(idx, jnp.int32).T
    wu = lax.bitcast_convert_type(w.T.astype(jnp.bfloat16),
                                  jnp.uint16).astype(jnp.uint32)
    wpack = lax.bitcast_convert_type(wu | (wu << 16), jnp.int32)

    s = jax.nn.softplus(s_raw) + 1e-6
    bterm = s * theta

    mesh = plsc.VectorSubcoreMesh(core_axis_name="c", subcore_axis_name="s")
    run = functools.partial(
        pl.kernel,
        mesh=mesh,
        compiler_params=pltpu.CompilerParams(needs_layout_passes=False),
        out_type=jax.ShapeDtypeStruct((B, OUT_DIM), jnp.float32),
        scratch_types=(
            [pltpu.VMEM((IN_DIM,), jnp.int32) for _ in range(PB)]
            + [pltpu.VMEM((OUT_DIM,), jnp.float32) for _ in range(2 * PB)]
            + [
                pltpu.VMEM((FAN_IN, OUT_DIM), jnp.int32),
                pltpu.VMEM((FAN_IN, OUT_DIM), jnp.int32),
                pltpu.VMEM((OUT_DIM,), jnp.float32),
                pltpu.VMEM((OUT_DIM,), jnp.float32),
                pltpu.SemaphoreType.DMA,
            ]
        ),
    )(_tec_body)
    return run(xpack, idxT, wpack, s, bterm)


# speedup vs baseline: 1.7072x; 1.7072x over previous
import functools

import jax
import jax.numpy as jnp
from jax import lax
from jax.experimental import pallas as pl
from jax.experimental.pallas import tpu as pltpu
from jax.experimental.pallas import tpu_sc as plsc

B = 1024
IN_DIM = 4096
OUT_DIM = 4096
FAN_IN = 8
L = 16
NC, NS = 2, 16
NW = NC * NS
RPW = B // NW
G = OUT_DIM // L
C = IN_DIM // L
PB = 4
NPASS = RPW // (2 * PB)


def _tec_body(x_hbm, idxT_hbm, wp_hbm, s_hbm, b_hbm, out_hbm,
              x0, x1, x2, x3, y0, y1, y2, y3, y4, y5, y6, y7,
              idxv, wv, sv, bv, sem):
    xp = (x0, x1, x2, x3)
    yr = (y0, y1, y2, y3, y4, y5, y6, y7)
    wid = lax.axis_index("s") * NC + lax.axis_index("c")
    base = wid * RPW
    pltpu.sync_copy(idxT_hbm, idxv)
    pltpu.sync_copy(wp_hbm, wv)
    pltpu.sync_copy(s_hbm, sv)
    pltpu.sync_copy(b_hbm, bv)

    def pass_body(p, carry):
        row = base + p * 2 * PB
        cps = [pltpu.async_copy(x_hbm.at[row + r], yr[r], sem)
               for r in range(2 * PB)]
        for c in cps:
            c.wait()

        def pack_body(c, c2):
            o = c * L
            for j in range(PB):
                pk = plsc.pack(yr[2 * j][pl.ds(o, L)],
                               yr[2 * j + 1][pl.ds(o, L)],
                               format=plsc.PackFormat.INTERLEAVED)
                xp[j][pl.ds(o, L)] = plsc.bitcast(pk, jnp.int32)
            return c2

        lax.fori_loop(0, C, pack_body, 0)

        def grp_body(g, c2):
            o = g * L
            acc = [None] * PB
            for k in range(FAN_IN):
                ivec = idxv[k, pl.ds(o, L)]
                wvec = plsc.bitcast(wv[k, pl.ds(o, L)], jnp.bfloat16)
                for j in range(PB):
                    gb = plsc.bitcast(plsc.load_gather(xp[j], [ivec]),
                                      jnp.bfloat16)
                    t = gb * wvec
                    acc[j] = t if k == 0 else acc[j] + t
            svec = sv[pl.ds(o, L)]
            bvec = bv[pl.ds(o, L)]
            for j in range(PB):
                lo, hi = plsc.unpack(acc[j],
                                     format=plsc.PackFormat.INTERLEAVED)
                for r, val in ((2 * j, lo), (2 * j + 1, hi)):
                    z = val * svec - bvec
                    yr[r][pl.ds(o, L)] = 1.0 / (1.0 + jnp.exp(-z))
            return c2

        lax.fori_loop(0, G, grp_body, 0)
        ocps = [pltpu.async_copy(yr[r], out_hbm.at[row + r], sem)
                for r in range(2 * PB)]
        for c in ocps:
            c.wait()
        return carry

    lax.fori_loop(0, NPASS, pass_body, 0)


def kernel(x, idx, w, theta, s_raw):
    idxT = jnp.asarray(idx, jnp.int32).T
    wu = lax.bitcast_convert_type(w.T.astype(jnp.bfloat16),
                                  jnp.uint16).astype(jnp.uint32)
    wpack = lax.bitcast_convert_type(wu | (wu << 16), jnp.int32)

    s = jax.nn.softplus(s_raw) + 1e-6
    bterm = s * theta

    mesh = plsc.VectorSubcoreMesh(core_axis_name="c", subcore_axis_name="s")
    run = functools.partial(
        pl.kernel,
        mesh=mesh,
        compiler_params=pltpu.CompilerParams(needs_layout_passes=False),
        out_type=jax.ShapeDtypeStruct((B, OUT_DIM), jnp.float32),
        scratch_types=(
            [pltpu.VMEM((IN_DIM,), jnp.int32) for _ in range(PB)]
            + [pltpu.VMEM((OUT_DIM,), jnp.float32) for _ in range(2 * PB)]
            + [
                pltpu.VMEM((FAN_IN, OUT_DIM), jnp.int32),
                pltpu.VMEM((FAN_IN, OUT_DIM), jnp.int32),
                pltpu.VMEM((OUT_DIM,), jnp.float32),
                pltpu.VMEM((OUT_DIM,), jnp.float32),
                pltpu.SemaphoreType.DMA,
            ]
        ),
    )(_tec_body)
    return run(x, idxT, wpack, s, bterm)


# speedup vs baseline: 2.3458x; 1.3740x over previous
import functools

import jax
import jax.numpy as jnp
from jax import lax
from jax.experimental import pallas as pl
from jax.experimental.pallas import tpu as pltpu
from jax.experimental.pallas import tpu_sc as plsc

B = 1024
IN_DIM = 4096
OUT_DIM = 4096
FAN_IN = 8
L = 16
NC, NS = 2, 16
NW = NC * NS
RPW = B // NW
G = OUT_DIM // L
C = IN_DIM // L
PB = 4
NPASS = RPW // (2 * PB)


def _tec_body(x_hbm, idxT_hbm, wp_hbm, s_hbm, b_hbm, out_hbm,
              x0, x1, x2, x3, y0, y1, y2, y3, y4, y5, y6, y7,
              idxv, wv, sv, bv, sem):
    xp = (x0, x1, x2, x3)
    yr = (y0, y1, y2, y3, y4, y5, y6, y7)
    wid = lax.axis_index("s") * NC + lax.axis_index("c")
    base = wid * RPW
    pltpu.sync_copy(idxT_hbm, idxv)
    pltpu.sync_copy(wp_hbm, wv)
    pltpu.sync_copy(s_hbm, sv)
    pltpu.sync_copy(b_hbm, bv)

    def pass_body(p, carry):
        row = base + p * 2 * PB
        cps = [pltpu.async_copy(x_hbm.at[row + r], yr[r], sem)
               for r in range(2 * PB)]
        for c in cps:
            c.wait()

        @plsc.parallel_loop(0, C, 1, unroll=4)
        def pack_body(c):
            o = c * L
            for j in range(PB):
                pk = plsc.pack(yr[2 * j][pl.ds(o, L)],
                               yr[2 * j + 1][pl.ds(o, L)],
                               format=plsc.PackFormat.INTERLEAVED)
                xp[j][pl.ds(o, L)] = plsc.bitcast(pk, jnp.int32)

        @plsc.parallel_loop(0, G, 1, unroll=2)
        def grp_body(g):
            o = g * L
            acc = [None] * PB
            for k in range(FAN_IN):
                ivec = idxv[k, pl.ds(o, L)]
                wvec = plsc.bitcast(wv[k, pl.ds(o, L)], jnp.bfloat16)
                for j in range(PB):
                    gb = plsc.bitcast(plsc.load_gather(xp[j], [ivec]),
                                      jnp.bfloat16)
                    t = gb * wvec
                    acc[j] = t if k == 0 else acc[j] + t
            svec = sv[pl.ds(o, L)]
            bvec = bv[pl.ds(o, L)]
            for j in range(PB):
                lo, hi = plsc.unpack(acc[j],
                                     format=plsc.PackFormat.INTERLEAVED)
                for r, val in ((2 * j, lo), (2 * j + 1, hi)):
                    z = val * svec - bvec
                    yr[r][pl.ds(o, L)] = 1.0 / (1.0 + jnp.exp(-z))
        ocps = [pltpu.async_copy(yr[r], out_hbm.at[row + r], sem)
                for r in range(2 * PB)]
        for c in ocps:
            c.wait()
        return carry

    lax.fori_loop(0, NPASS, pass_body, 0)


def kernel(x, idx, w, theta, s_raw):
    idxT = jnp.asarray(idx, jnp.int32).T
    wu = lax.bitcast_convert_type(w.T.astype(jnp.bfloat16),
                                  jnp.uint16).astype(jnp.uint32)
    wpack = lax.bitcast_convert_type(wu | (wu << 16), jnp.int32)

    s = jax.nn.softplus(s_raw) + 1e-6
    bterm = s * theta

    mesh = plsc.VectorSubcoreMesh(core_axis_name="c", subcore_axis_name="s")
    run = functools.partial(
        pl.kernel,
        mesh=mesh,
        compiler_params=pltpu.CompilerParams(needs_layout_passes=False),
        out_type=jax.ShapeDtypeStruct((B, OUT_DIM), jnp.float32),
        scratch_types=(
            [pltpu.VMEM((IN_DIM,), jnp.int32) for _ in range(PB)]
            + [pltpu.VMEM((OUT_DIM,), jnp.float32) for _ in range(2 * PB)]
            + [
                pltpu.VMEM((FAN_IN, OUT_DIM), jnp.int32),
                pltpu.VMEM((FAN_IN, OUT_DIM), jnp.int32),
                pltpu.VMEM((OUT_DIM,), jnp.float32),
                pltpu.VMEM((OUT_DIM,), jnp.float32),
                pltpu.SemaphoreType.DMA,
            ]
        ),
    )(_tec_body)
    return run(x, idxT, wpack, s, bterm)


# speedup vs baseline: 2.3529x; 1.0031x over previous
import functools

import jax
import jax.numpy as jnp
from jax import lax
from jax.experimental import pallas as pl
from jax.experimental.pallas import tpu as pltpu
from jax.experimental.pallas import tpu_sc as plsc

B = 1024
IN_DIM = 4096
OUT_DIM = 4096
FAN_IN = 8
L = 16
NC, NS = 2, 16
NW = NC * NS
RPW = B // NW
G = OUT_DIM // L
C = IN_DIM // L
PB = 4
NPASS = RPW // (2 * PB)


def _tec_body(x_hbm, idxT_hbm, wp_hbm, s_hbm, b_hbm, out_hbm,
              x0, x1, x2, x3, y0, y1, y2, y3, y4, y5, y6, y7,
              idxv, wv, sv, bv, sem):
    xp = (x0, x1, x2, x3)
    yr = (y0, y1, y2, y3, y4, y5, y6, y7)
    wid = lax.axis_index("s") * NC + lax.axis_index("c")
    base = wid * RPW
    pltpu.sync_copy(idxT_hbm, idxv)
    pltpu.sync_copy(wp_hbm, wv)
    pltpu.sync_copy(s_hbm, sv)
    pltpu.sync_copy(b_hbm, bv)

    def pass_body(p, carry):
        row = base + p * 2 * PB
        cps = [pltpu.async_copy(x_hbm.at[row + r], yr[r], sem)
               for r in range(2 * PB)]
        for c in cps:
            c.wait()

        @plsc.parallel_loop(0, C, 1, unroll=4)
        def pack_body(c):
            o = c * L
            for j in range(PB):
                pk = plsc.pack(yr[2 * j][pl.ds(o, L)],
                               yr[2 * j + 1][pl.ds(o, L)],
                               format=plsc.PackFormat.INTERLEAVED)
                xp[j][pl.ds(o, L)] = plsc.bitcast(pk, jnp.int32)

        @plsc.parallel_loop(0, G, 1, unroll=4)
        def grp_body(g):
            o = g * L
            acc = [None] * PB
            for k in range(FAN_IN):
                ivec = idxv[k, pl.ds(o, L)]
                wvec = plsc.bitcast(wv[k, pl.ds(o, L)], jnp.bfloat16)
                for j in range(PB):
                    gb = plsc.bitcast(plsc.load_gather(xp[j], [ivec]),
                                      jnp.bfloat16)
                    t = gb * wvec
                    acc[j] = t if k == 0 else acc[j] + t
            svec = sv[pl.ds(o, L)]
            bvec = bv[pl.ds(o, L)]
            for j in range(PB):
                lo, hi = plsc.unpack(acc[j],
                                     format=plsc.PackFormat.INTERLEAVED)
                for r, val in ((2 * j, lo), (2 * j + 1, hi)):
                    z = val * svec - bvec
                    yr[r][pl.ds(o, L)] = 1.0 / (1.0 + jnp.exp(-z))
        ocps = [pltpu.async_copy(yr[r], out_hbm.at[row + r], sem)
                for r in range(2 * PB)]
        for c in ocps:
            c.wait()
        return carry

    lax.fori_loop(0, NPASS, pass_body, 0)


def kernel(x, idx, w, theta, s_raw):
    idxT = jnp.asarray(idx, jnp.int32).T
    wu = lax.bitcast_convert_type(w.T.astype(jnp.bfloat16),
                                  jnp.uint16).astype(jnp.uint32)
    wpack = lax.bitcast_convert_type(wu | (wu << 16), jnp.int32)

    s = jax.nn.softplus(s_raw) + 1e-6
    bterm = s * theta

    mesh = plsc.VectorSubcoreMesh(core_axis_name="c", subcore_axis_name="s")
    run = functools.partial(
        pl.kernel,
        mesh=mesh,
        compiler_params=pltpu.CompilerParams(needs_layout_passes=False),
        out_type=jax.ShapeDtypeStruct((B, OUT_DIM), jnp.float32),
        scratch_types=(
            [pltpu.VMEM((IN_DIM,), jnp.int32) for _ in range(PB)]
            + [pltpu.VMEM((OUT_DIM,), jnp.float32) for _ in range(2 * PB)]
            + [
                pltpu.VMEM((FAN_IN, OUT_DIM), jnp.int32),
                pltpu.VMEM((FAN_IN, OUT_DIM), jnp.int32),
                pltpu.VMEM((OUT_DIM,), jnp.float32),
                pltpu.VMEM((OUT_DIM,), jnp.float32),
                pltpu.SemaphoreType.DMA,
            ]
        ),
    )(_tec_body)
    return run(x, idxT, wpack, s, bterm)


# speedup vs baseline: 2.4575x; 1.0445x over previous
import functools

import jax
import jax.numpy as jnp
from jax import lax
from jax.experimental import pallas as pl
from jax.experimental.pallas import tpu as pltpu
from jax.experimental.pallas import tpu_sc as plsc

B = 1024
IN_DIM = 4096
OUT_DIM = 4096
FAN_IN = 8
L = 16
NC, NS = 2, 16
NW = NC * NS
RPW = B // NW
G = OUT_DIM // L
C = IN_DIM // L
PB = 4
NPASS = RPW // (2 * PB)


def _tec_body(x_hbm, ip_hbm, wp_hbm, b_hbm, out_hbm,
              x0, x1, x2, x3, y0, y1, y2, y3, y4, y5, y6, y7,
              ipv, wv, bv, sem):
    xp = (x0, x1, x2, x3)
    yr = (y0, y1, y2, y3, y4, y5, y6, y7)
    wid = lax.axis_index("s") * NC + lax.axis_index("c")
    base = wid * RPW
    pltpu.sync_copy(ip_hbm, ipv)
    pltpu.sync_copy(wp_hbm, wv)
    pltpu.sync_copy(b_hbm, bv)

    def pass_body(p, carry):
        row = base + p * 2 * PB
        cps = [pltpu.async_copy(x_hbm.at[row + r], yr[r], sem)
               for r in range(2 * PB)]
        for c in cps:
            c.wait()

        @plsc.parallel_loop(0, C, 1, unroll=4)
        def pack_body(c):
            o = c * L
            for j in range(PB):
                pk = plsc.pack(yr[2 * j][pl.ds(o, L)],
                               yr[2 * j + 1][pl.ds(o, L)],
                               format=plsc.PackFormat.INTERLEAVED)
                xp[j][pl.ds(o, L)] = plsc.bitcast(pk, jnp.int32)

        @plsc.parallel_loop(0, G, 1, unroll=2)
        def grp_body(g):
            o = g * L
            acc = [None] * PB
            for t in range(FAN_IN // 2):
                iw = plsc.bitcast(ipv[t, pl.ds(o, L)], jnp.int16)
                iv0, iv1 = plsc.unpack(iw, format=plsc.PackFormat.INTERLEAVED)
                for k, ivec in ((2 * t, iv0), (2 * t + 1, iv1)):
                    wvec = plsc.bitcast(wv[k, pl.ds(o, L)], jnp.bfloat16)
                    for j in range(PB):
                        gb = plsc.bitcast(plsc.load_gather(xp[j], [ivec]),
                                          jnp.bfloat16)
                        t2 = gb * wvec
                        acc[j] = t2 if k == 0 else acc[j] + t2
            bvec = bv[pl.ds(o, L)]
            for j in range(PB):
                lo, hi = plsc.unpack(acc[j],
                                     format=plsc.PackFormat.INTERLEAVED)
                for r, val in ((2 * j, lo), (2 * j + 1, hi)):
                    yr[r][pl.ds(o, L)] = 1.0 / (1.0 + jnp.exp(bvec - val))

        ocps = [pltpu.async_copy(yr[r], out_hbm.at[row + r], sem)
                for r in range(2 * PB)]
        for c in ocps:
            c.wait()
        return carry

    lax.fori_loop(0, NPASS, pass_body, 0)


def kernel(x, idx, w, theta, s_raw):
    s = jax.nn.softplus(s_raw) + 1e-6
    bterm = s * theta

    iu = jnp.asarray(idx, jnp.uint32).T
    ipack = lax.bitcast_convert_type(iu[0::2] | (iu[1::2] << 16), jnp.int32)

    wu = lax.bitcast_convert_type((w * s[:, None]).T.astype(jnp.bfloat16),
                                  jnp.uint16).astype(jnp.uint32)
    wpack = lax.bitcast_convert_type(wu | (wu << 16), jnp.int32)

    mesh = plsc.VectorSubcoreMesh(core_axis_name="c", subcore_axis_name="s")
    run = functools.partial(
        pl.kernel,
        mesh=mesh,
        compiler_params=pltpu.CompilerParams(needs_layout_passes=False),
        out_type=jax.ShapeDtypeStruct((B, OUT_DIM), jnp.float32),
        scratch_types=(
            [pltpu.VMEM((IN_DIM,), jnp.int32) for _ in range(PB)]
            + [pltpu.VMEM((OUT_DIM,), jnp.float32) for _ in range(2 * PB)]
            + [
                pltpu.VMEM((FAN_IN // 2, OUT_DIM), jnp.int32),
                pltpu.VMEM((FAN_IN, OUT_DIM), jnp.int32),
                pltpu.VMEM((OUT_DIM,), jnp.float32),
                pltpu.SemaphoreType.DMA,
            ]
        ),
    )(_tec_body)
    return run(x, ipack, wpack, bterm)


# speedup vs baseline: 2.5129x; 1.0225x over previous
import functools

import jax
import jax.numpy as jnp
from jax import lax
from jax.experimental import pallas as pl
from jax.experimental.pallas import tpu as pltpu
from jax.experimental.pallas import tpu_sc as plsc

B = 1024
IN_DIM = 4096
OUT_DIM = 4096
FAN_IN = 8
L = 16
NC, NS = 2, 16
NW = NC * NS
RPW = B // NW
G = OUT_DIM // L
C = IN_DIM // L
PB = 4
NPASS = RPW // (2 * PB)


def _tec_body(x_hbm, ip_hbm, wp_hbm, b_hbm, out_hbm,
              x0, x1, x2, x3, y0, y1, y2, y3, y4, y5, y6, y7,
              ipv, wv, bv, sem):
    xp = (x0, x1, x2, x3)
    yr = (y0, y1, y2, y3, y4, y5, y6, y7)
    wid = lax.axis_index("s") * NC + lax.axis_index("c")
    base = wid * RPW
    pltpu.sync_copy(ip_hbm, ipv)
    pltpu.sync_copy(wp_hbm, wv)
    pltpu.sync_copy(b_hbm, bv)

    def pass_body(p, carry):
        row = base + p * 2 * PB
        cps = [pltpu.async_copy(x_hbm.at[row + r], yr[r], sem)
               for r in range(2 * PB)]
        for c in cps:
            c.wait()

        @plsc.parallel_loop(0, C, 1, unroll=4)
        def pack_body(c):
            o = c * L
            for j in range(PB):
                pk = plsc.pack(yr[2 * j][pl.ds(o, L)],
                               yr[2 * j + 1][pl.ds(o, L)],
                               format=plsc.PackFormat.INTERLEAVED)
                xp[j][pl.ds(o, L)] = plsc.bitcast(pk, jnp.int32)

        @plsc.parallel_loop(0, G, 1, unroll=2)
        def grp_body(g):
            o = g * L
            acc = [None] * PB
            for t in range(FAN_IN // 2):
                iw = plsc.bitcast(ipv[t, pl.ds(o, L)], jnp.int16)
                iv0, iv1 = plsc.unpack(iw, format=plsc.PackFormat.INTERLEAVED)
                ww = plsc.bitcast(wv[t, pl.ds(o, L)], jnp.bfloat16)
                wa, wb = plsc.unpack(ww, format=plsc.PackFormat.INTERLEAVED)
                for k, ivec, wf in ((2 * t, iv0, wa), (2 * t + 1, iv1, wb)):
                    wvec = plsc.pack(wf, wf,
                                     format=plsc.PackFormat.INTERLEAVED)
                    for j in range(PB):
                        gb = plsc.bitcast(plsc.load_gather(xp[j], [ivec]),
                                          jnp.bfloat16)
                        t2 = gb * wvec
                        acc[j] = t2 if k == 0 else acc[j] + t2
            bvec = bv[pl.ds(o, L)]
            for j in range(PB):
                lo, hi = plsc.unpack(acc[j],
                                     format=plsc.PackFormat.INTERLEAVED)
                for r, val in ((2 * j, lo), (2 * j + 1, hi)):
                    yr[r][pl.ds(o, L)] = 1.0 / (1.0 + jnp.exp(bvec - val))

        ocps = [pltpu.async_copy(yr[r], out_hbm.at[row + r], sem)
                for r in range(2 * PB)]
        for c in ocps:
            c.wait()
        return carry

    lax.fori_loop(0, NPASS, pass_body, 0)


def kernel(x, idx, w, theta, s_raw):
    s = jax.nn.softplus(s_raw) + 1e-6
    bterm = s * theta

    iu = jnp.asarray(idx, jnp.uint32).T
    ipack = lax.bitcast_convert_type(iu[0::2] | (iu[1::2] << 16), jnp.int32)

    wu = lax.bitcast_convert_type((w * s[:, None]).T.astype(jnp.bfloat16),
                                  jnp.uint16).astype(jnp.uint32)
    wpack = lax.bitcast_convert_type(wu[0::2] | (wu[1::2] << 16), jnp.int32)

    mesh = plsc.VectorSubcoreMesh(core_axis_name="c", subcore_axis_name="s")
    run = functools.partial(
        pl.kernel,
        mesh=mesh,
        compiler_params=pltpu.CompilerParams(needs_layout_passes=False),
        out_type=jax.ShapeDtypeStruct((B, OUT_DIM), jnp.float32),
        scratch_types=(
            [pltpu.VMEM((IN_DIM,), jnp.int32) for _ in range(PB)]
            + [pltpu.VMEM((OUT_DIM,), jnp.float32) for _ in range(2 * PB)]
            + [
                pltpu.VMEM((FAN_IN // 2, OUT_DIM), jnp.int32),
                pltpu.VMEM((FAN_IN // 2, OUT_DIM), jnp.int32),
                pltpu.VMEM((OUT_DIM,), jnp.float32),
                pltpu.SemaphoreType.DMA,
            ]
        ),
    )(_tec_body)
    return run(x, ipack, wpack, bterm)
